# Initial kernel scaffold; baseline (speedup 1.0000x reference)
import jax
import jax.numpy as jnp
from jax.experimental import pallas as pl


def kernel(h, edge_index, W, b):
    raise NotImplementedError("write your pallas kernel here")


# speedup vs baseline: 37.1920x; 37.1920x over previous
import functools

import jax
import jax.numpy as jnp
from jax import lax
from jax.experimental import pallas as pl
from jax.experimental.pallas import tpu as pltpu
from jax.experimental.pallas import tpu_sc as plsc

N = 10000
E = 320000
D = 128
NC = 2
NS = 16
NW = NC * NS
EPW = E // NW
L = 16


def _pq_body(h_ref, w_ref, b_ref, pq_ref):
    hmat = h_ref[:, :]
    w2 = w_ref[:, :]
    pq = lax.dot_general(w2, hmat, (((1,), (1,)), ((), ())),
                         preferred_element_type=jnp.float32)
    rowid = lax.broadcasted_iota(jnp.int32, (2, N), 0)
    pq_ref[:, :] = pq + jnp.where(rowid == 0, b_ref[0], 0.0)


def _norm_body(s_ref, mm_ref, o_ref):
    mn = jnp.min(mm_ref[:, 0])
    mx = jnp.max(mm_ref[:, 1])
    scale = 1.0 / (mx - mn)
    o_ref[:, :] = (s_ref[:, :] - mn) * scale


def _sc_body(pq_hbm, ei_hbm, s_hbm, mm_hbm, p_v, q_v, src_v, dst_v, s_v, mm_v):
    wid = lax.axis_index("s") * NC + lax.axis_index("c")
    base = wid * EPW
    pltpu.sync_copy(pq_hbm.at[0], p_v)
    pltpu.sync_copy(pq_hbm.at[1], q_v)
    pltpu.sync_copy(ei_hbm.at[pl.ds(base, EPW)], src_v)
    pltpu.sync_copy(ei_hbm.at[pl.ds(E + base, EPW)], dst_v)

    def body(i, carry):
        mn, mx = carry
        off = pl.multiple_of(i * L, L)
        si = src_v[pl.ds(off, L)]
        di = dst_v[pl.ds(off, L)]
        pv = plsc.load_gather(p_v, [si])
        qv = plsc.load_gather(q_v, [di])
        sv = pv + qv
        s_v[pl.ds(off, L)] = sv
        return jnp.minimum(mn, sv), jnp.maximum(mx, sv)

    init = (jnp.full((L,), jnp.inf, jnp.float32),
            jnp.full((L,), -jnp.inf, jnp.float32))
    mn, mx = lax.fori_loop(0, EPW // L, body, init)
    mn_s = jnp.min(mn)
    mx_s = jnp.max(mx)
    lane = lax.iota(jnp.int32, L)
    mm_v[...] = jnp.where(lane == 1, mx_s, mn_s)
    pltpu.sync_copy(s_v, s_hbm.at[pl.ds(base, EPW)])
    pltpu.sync_copy(mm_v, mm_hbm.at[wid])


def kernel(h, edge_index, W, b):
    w2 = W.reshape(2, D)

    pq = pl.pallas_call(
        _pq_body,
        out_shape=jax.ShapeDtypeStruct((2, N), jnp.float32),
        in_specs=[
            pl.BlockSpec(),
            pl.BlockSpec(),
            pl.BlockSpec(memory_space=pltpu.SMEM),
        ],
    )(h, w2, b)

    sc = pl.kernel(
        _sc_body,
        out_type=(jax.ShapeDtypeStruct((E,), jnp.float32),
                  jax.ShapeDtypeStruct((NW, L), jnp.float32)),
        mesh=plsc.VectorSubcoreMesh(core_axis_name="c", subcore_axis_name="s",
                                    num_cores=NC, num_subcores=NS),
        compiler_params=pltpu.CompilerParams(needs_layout_passes=False),
        scratch_types=[
            pltpu.VMEM((N,), jnp.float32),
            pltpu.VMEM((N,), jnp.float32),
            pltpu.VMEM((EPW,), jnp.int32),
            pltpu.VMEM((EPW,), jnp.int32),
            pltpu.VMEM((EPW,), jnp.float32),
            pltpu.VMEM((L,), jnp.float32),
        ],
    )
    s, mm = sc(pq, edge_index.reshape(2 * E))

    o = pl.pallas_call(
        _norm_body,
        out_shape=jax.ShapeDtypeStruct((E // D, D), jnp.float32),
    )(s.reshape(E // D, D), mm)
    return o.reshape(E, 1)


# speedup vs baseline: 41.2019x; 1.1078x over previous
import jax
import jax.numpy as jnp
from jax import lax
from jax.experimental import pallas as pl
from jax.experimental.pallas import tpu as pltpu
from jax.experimental.pallas import tpu_sc as plsc

N = 10000
E = 320000
D = 128
NC = 2
NS = 16
NW = NC * NS
EPW = E // NW
L = 16


def _pq_body(h_ref, w_ref, b_ref, pq_ref):
    hmat = h_ref[:, :]
    w2 = w_ref[:, :]
    pq = lax.dot_general(w2, hmat, (((1,), (1,)), ((), ())),
                         preferred_element_type=jnp.float32)
    rowid = lax.broadcasted_iota(jnp.int32, (2, N), 0)
    pq_ref[:, :] = pq + jnp.where(rowid == 0, b_ref[0], 0.0)


def _norm_body(s_ref, o_ref):
    s = s_ref[:, :]
    mn = jnp.min(s)
    mx = jnp.max(s)
    scale = 1.0 / (mx - mn)
    o_ref[:, :] = (s - mn) * scale


def _sc_body(pq_hbm, ei_hbm, s_hbm, p_v, q_v, src_v, dst_v, s_v, sem):
    wid = lax.axis_index("s") * NC + lax.axis_index("c")
    base = wid * EPW
    c1 = pltpu.async_copy(pq_hbm.at[0], p_v, sem)
    c2 = pltpu.async_copy(pq_hbm.at[1], q_v, sem)
    c3 = pltpu.async_copy(ei_hbm.at[pl.ds(base, EPW)], src_v, sem)
    c4 = pltpu.async_copy(ei_hbm.at[pl.ds(E + base, EPW)], dst_v, sem)
    c1.wait()
    c2.wait()
    c3.wait()
    c4.wait()

    @plsc.parallel_loop(0, EPW // L, unroll=8)
    def _(i):
        off = pl.multiple_of(i * L, L)
        si = src_v[pl.ds(off, L)]
        di = dst_v[pl.ds(off, L)]
        s_v[pl.ds(off, L)] = (plsc.load_gather(p_v, [si])
                              + plsc.load_gather(q_v, [di]))

    pltpu.sync_copy(s_v, s_hbm.at[pl.ds(base, EPW)])


def kernel(h, edge_index, W, b):
    w2 = W.reshape(2, D)

    pq = pl.pallas_call(
        _pq_body,
        out_shape=jax.ShapeDtypeStruct((2, N), jnp.float32),
        in_specs=[
            pl.BlockSpec(),
            pl.BlockSpec(),
            pl.BlockSpec(memory_space=pltpu.SMEM),
        ],
    )(h, w2, b)

    sc = pl.kernel(
        _sc_body,
        out_type=jax.ShapeDtypeStruct((E,), jnp.float32),
        mesh=plsc.VectorSubcoreMesh(core_axis_name="c", subcore_axis_name="s",
                                    num_cores=NC, num_subcores=NS),
        compiler_params=pltpu.CompilerParams(needs_layout_passes=False),
        scratch_types=[
            pltpu.VMEM((N,), jnp.float32),
            pltpu.VMEM((N,), jnp.float32),
            pltpu.VMEM((EPW,), jnp.int32),
            pltpu.VMEM((EPW,), jnp.int32),
            pltpu.VMEM((EPW,), jnp.float32),
            pltpu.SemaphoreType.DMA,
        ],
    )
    s = sc(pq, edge_index.reshape(2 * E))

    o = pl.pallas_call(
        _norm_body,
        out_shape=jax.ShapeDtypeStruct((E // D, D), jnp.float32),
    )(s.reshape(E // D, D))
    return o.reshape(E, 1)


# speedup vs baseline: 53.9745x; 1.3100x over previous
import jax
import jax.numpy as jnp
from jax import lax
from jax.experimental import pallas as pl
from jax.experimental.pallas import tpu as pltpu
from jax.experimental.pallas import tpu_sc as plsc

N = 10000
E = 320000
D = 128
NC = 2
NS = 16
NW = NC * NS
L = 16
CHUNK = 128
NCH = E // CHUNK
CHW = 79
EW = CHW * CHUNK
LAST_C0 = NCH - CHW


def _pq_body(h_ref, w_ref, b_ref, pq_ref):
    hmat = h_ref[:, :]
    w2 = w_ref[:, :]
    pq = lax.dot_general(w2, hmat, (((1,), (1,)), ((), ())),
                         preferred_element_type=jnp.float32)
    rowid = lax.broadcasted_iota(jnp.int32, (2, N), 0)
    pq_ref[:, :] = pq + jnp.where(rowid == 0, b_ref[0], 0.0)


def _norm_body(s_ref, o_ref):
    s = s_ref[...]
    mn = jnp.min(s)
    mx = jnp.max(s)
    scale = 1.0 / (mx - mn)
    o_ref[...] = (s - mn) * scale


def _sc_body(pq_hbm, ei_hbm, s_hbm, p_v, q_v, ei_v, s_v, sem):
    wid = lax.axis_index("s") * NC + lax.axis_index("c")
    c0 = (wid * LAST_C0) // (NW - 1)
    base = c0 * CHUNK
    c1 = pltpu.async_copy(pq_hbm.at[0], p_v, sem)
    c2 = pltpu.async_copy(pq_hbm.at[1], q_v, sem)
    c3 = pltpu.async_copy(ei_hbm.at[:, pl.ds(base, EW)], ei_v, sem)
    c1.wait()
    c2.wait()
    c3.wait()

    @plsc.parallel_loop(0, EW // L, unroll=8)
    def _(i):
        off = pl.multiple_of(i * L, L)
        si = ei_v[0, pl.ds(off, L)]
        di = ei_v[1, pl.ds(off, L)]
        s_v[pl.ds(off, L)] = (plsc.load_gather(p_v, [si])
                              + plsc.load_gather(q_v, [di]))

    pltpu.sync_copy(s_v, s_hbm.at[0, pl.ds(base, EW)])


def kernel(h, edge_index, W, b):
    w2 = W.reshape(2, D)

    pq = pl.pallas_call(
        _pq_body,
        out_shape=jax.ShapeDtypeStruct((2, N), jnp.float32),
        in_specs=[
            pl.BlockSpec(),
            pl.BlockSpec(),
            pl.BlockSpec(memory_space=pltpu.SMEM),
        ],
    )(h, w2, b)

    sc = pl.kernel(
        _sc_body,
        out_type=jax.ShapeDtypeStruct((1, E), jnp.float32),
        mesh=plsc.VectorSubcoreMesh(core_axis_name="c", subcore_axis_name="s",
                                    num_cores=NC, num_subcores=NS),
        compiler_params=pltpu.CompilerParams(needs_layout_passes=False),
        scratch_types=[
            pltpu.VMEM((N,), jnp.float32),
            pltpu.VMEM((N,), jnp.float32),
            pltpu.VMEM((2, EW), jnp.int32),
            pltpu.VMEM((EW,), jnp.float32),
            pltpu.SemaphoreType.DMA,
        ],
    )
    s = sc(pq, edge_index)

    o = pl.pallas_call(
        _norm_body,
        out_shape=jax.ShapeDtypeStruct((1, E), jnp.float32),
    )(s)
    return o.reshape(E, 1)


# speedup vs baseline: 55.5026x; 1.0283x over previous
import jax
import jax.numpy as jnp
from jax import lax
from jax.experimental import pallas as pl
from jax.experimental.pallas import tpu as pltpu
from jax.experimental.pallas import tpu_sc as plsc

N = 10000
E = 320000
D = 128
NC = 2
NS = 16
NW = NC * NS
L = 16
CHUNK = 128
NCH = E // CHUNK
CHW = 79
EW = CHW * CHUNK
LAST_C0 = NCH - CHW


def _pq_body(h_ref, w_ref, b_ref, pq_ref):
    hmat = h_ref[:, :]
    w2 = w_ref[:, :]
    pq = lax.dot_general(w2, hmat, (((1,), (1,)), ((), ())),
                         preferred_element_type=jnp.float32)
    rowid = lax.broadcasted_iota(jnp.int32, (2, N), 0)
    pq_ref[:, :] = pq + jnp.where(rowid == 0, b_ref[0], 0.0)


def _norm_body(s_ref, mm_ref, o_ref):
    mn = jnp.min(mm_ref[:, 0])
    mx = jnp.max(mm_ref[:, 1])
    scale = 1.0 / (mx - mn)
    o_ref[...] = (s_ref[...] - mn) * scale


def _sc_body(pq_hbm, ei_hbm, s_hbm, mm_hbm, p_v, q_v, ei_v, s_v, mm_v, sem):
    wid = lax.axis_index("s") * NC + lax.axis_index("c")
    c0 = (wid * LAST_C0) // (NW - 1)
    base = c0 * CHUNK
    c1 = pltpu.async_copy(pq_hbm.at[0], p_v, sem)
    c2 = pltpu.async_copy(pq_hbm.at[1], q_v, sem)
    c3 = pltpu.async_copy(ei_hbm.at[:, pl.ds(base, EW)], ei_v, sem)
    c1.wait()
    c2.wait()
    c3.wait()

    init = (jnp.full((L,), jnp.inf, jnp.float32),
            jnp.full((L,), -jnp.inf, jnp.float32))

    @plsc.parallel_loop(0, EW // L, unroll=16, carry=init)
    def _(i, carry):
        mn, mx = carry
        off = pl.multiple_of(i * L, L)
        si = ei_v[0, pl.ds(off, L)]
        di = ei_v[1, pl.ds(off, L)]
        sv = plsc.load_gather(p_v, [si]) + plsc.load_gather(q_v, [di])
        s_v[pl.ds(off, L)] = sv
        return jnp.minimum(mn, sv), jnp.maximum(mx, sv)

    mn, mx = _
    lane = lax.iota(jnp.int32, L)
    mm_v[...] = jnp.where(lane == 1, jnp.max(mx), jnp.min(mn))
    pltpu.sync_copy(s_v, s_hbm.at[0, pl.ds(base, EW)])
    pltpu.sync_copy(mm_v, mm_hbm.at[wid])


def kernel(h, edge_index, W, b):
    w2 = W.reshape(2, D)

    pq = pl.pallas_call(
        _pq_body,
        out_shape=jax.ShapeDtypeStruct((2, N), jnp.float32),
        in_specs=[
            pl.BlockSpec(),
            pl.BlockSpec(),
            pl.BlockSpec(memory_space=pltpu.SMEM),
        ],
    )(h, w2, b)

    sc = pl.kernel(
        _sc_body,
        out_type=(jax.ShapeDtypeStruct((1, E), jnp.float32),
                  jax.ShapeDtypeStruct((NW, L), jnp.float32)),
        mesh=plsc.VectorSubcoreMesh(core_axis_name="c", subcore_axis_name="s",
                                    num_cores=NC, num_subcores=NS),
        compiler_params=pltpu.CompilerParams(needs_layout_passes=False),
        scratch_types=[
            pltpu.VMEM((N,), jnp.float32),
            pltpu.VMEM((N,), jnp.float32),
            pltpu.VMEM((2, EW), jnp.int32),
            pltpu.VMEM((EW,), jnp.float32),
            pltpu.VMEM((L,), jnp.float32),
            pltpu.SemaphoreType.DMA,
        ],
    )
    s, mm = sc(pq, edge_index)

    o = pl.pallas_call(
        _norm_body,
        out_shape=jax.ShapeDtypeStruct((1, E), jnp.float32),
    )(s, mm)
    return o.reshape(E, 1)


# speedup vs baseline: 55.9448x; 1.0080x over previous
import jax
import jax.numpy as jnp
from jax import lax
from jax.experimental import pallas as pl
from jax.experimental.pallas import tpu as pltpu
from jax.experimental.pallas import tpu_sc as plsc

N = 10000
E = 320000
D = 128
NC = 2
NS = 16
NW = NC * NS
L = 16
CHUNK = 128
NCH = E // CHUNK
CHW = 79
EW = CHW * CHUNK
LAST_C0 = NCH - CHW


def _pq_body(h_ref, w_ref, b_ref, pq_ref):
    hmat = h_ref[:, :]
    w2 = w_ref[:, :]
    pq = lax.dot_general(w2, hmat, (((1,), (1,)), ((), ())),
                         preferred_element_type=jnp.float32)
    rowid = lax.broadcasted_iota(jnp.int32, (2, N), 0)
    pq_ref[:, :] = pq + jnp.where(rowid == 0, b_ref[0], 0.0)


def _norm_body(s_ref, mm_ref, o_ref):
    mn = jnp.min(mm_ref[:, 0])
    mx = jnp.max(mm_ref[:, 1])
    scale = 1.0 / (mx - mn)
    o_ref[...] = (s_ref[...] - mn) * scale


def _sc_body(pq_hbm, ei_hbm, s_hbm, mm_hbm, p_v, q_v, ei_v, s_v, mm_v, sem):
    wid = lax.axis_index("s") * NC + lax.axis_index("c")
    c0 = (wid * LAST_C0) // (NW - 1)
    base = c0 * CHUNK
    c1 = pltpu.async_copy(pq_hbm.at[0], p_v, sem)
    c2 = pltpu.async_copy(pq_hbm.at[1], q_v, sem)
    c3 = pltpu.async_copy(ei_hbm.at[:, pl.ds(base, EW)], ei_v, sem)
    c1.wait()
    c2.wait()
    c3.wait()

    init = (jnp.full((L,), jnp.inf, jnp.float32),
            jnp.full((L,), -jnp.inf, jnp.float32))

    @plsc.parallel_loop(0, EW // L, unroll=4, carry=init)
    def _(i, carry):
        mn, mx = carry
        off = pl.multiple_of(i * L, L)
        si = ei_v[0, pl.ds(off, L)]
        di = ei_v[1, pl.ds(off, L)]
        sv = plsc.load_gather(p_v, [si]) + plsc.load_gather(q_v, [di])
        s_v[pl.ds(off, L)] = sv
        return jnp.minimum(mn, sv), jnp.maximum(mx, sv)

    mn, mx = _
    lane = lax.iota(jnp.int32, L)
    mm_v[...] = jnp.where(lane == 1, jnp.max(mx), jnp.min(mn))
    pltpu.sync_copy(s_v, s_hbm.at[0, pl.ds(base, EW)])
    pltpu.sync_copy(mm_v, mm_hbm.at[wid])


def kernel(h, edge_index, W, b):
    w2 = W.reshape(2, D)

    pq = pl.pallas_call(
        _pq_body,
        out_shape=jax.ShapeDtypeStruct((2, N), jnp.float32),
        in_specs=[
            pl.BlockSpec(),
            pl.BlockSpec(),
            pl.BlockSpec(memory_space=pltpu.SMEM),
        ],
    )(h, w2, b)

    sc = pl.kernel(
        _sc_body,
        out_type=(jax.ShapeDtypeStruct((1, E), jnp.float32),
                  jax.ShapeDtypeStruct((NW, L), jnp.float32)),
        mesh=plsc.VectorSubcoreMesh(core_axis_name="c", subcore_axis_name="s",
                                    num_cores=NC, num_subcores=NS),
        compiler_params=pltpu.CompilerParams(needs_layout_passes=False),
        scratch_types=[
            pltpu.VMEM((N,), jnp.float32),
            pltpu.VMEM((N,), jnp.float32),
            pltpu.VMEM((2, EW), jnp.int32),
            pltpu.VMEM((EW,), jnp.float32),
            pltpu.VMEM((L,), jnp.float32),
            pltpu.SemaphoreType.DMA,
        ],
    )
    s, mm = sc(pq, edge_index)

    o = pl.pallas_call(
        _norm_body,
        out_shape=jax.ShapeDtypeStruct((1, E), jnp.float32),
    )(s, mm)
    return o.reshape(E, 1)


# speedup vs baseline: 56.9536x; 1.0180x over previous
import jax
import jax.numpy as jnp
from jax import lax
from jax.experimental import pallas as pl
from jax.experimental.pallas import tpu as pltpu
from jax.experimental.pallas import tpu_sc as plsc

N = 10000
E = 320000
D = 128
NC = 2
NS = 16
NW = NC * NS
L = 16
CHUNK = 128
NCH = E // CHUNK
CHW = 79
EW = CHW * CHUNK
LAST_C0 = NCH - CHW


def _pq_body(h_ref, w_ref, b_ref, pq_ref):
    hmat = h_ref[:, :]
    w2 = w_ref[:, :]
    pq = lax.dot_general(w2, hmat, (((1,), (1,)), ((), ())),
                         preferred_element_type=jnp.float32)
    rowid = lax.broadcasted_iota(jnp.int32, (2, N), 0)
    pq_ref[:, :] = pq + jnp.where(rowid == 0, b_ref[0], 0.0)


def _norm_body(s_ref, mm_ref, o_ref):
    mn = jnp.min(mm_ref[:, 0])
    mx = jnp.max(mm_ref[:, 1])
    scale = 1.0 / (mx - mn)
    o_ref[...] = (s_ref[...] - mn) * scale


SEG1 = 40 * CHUNK
SEG2 = EW - SEG1


def _sc_body(pq_hbm, ei_hbm, s_hbm, mm_hbm, p_v, q_v, ei_v, s_v, mm_v,
             sem_a, sem_b, sem_w):
    wid = lax.axis_index("s") * NC + lax.axis_index("c")
    c0 = (wid * LAST_C0) // (NW - 1)
    base = c0 * CHUNK
    ca1 = pltpu.async_copy(ei_hbm.at[:, pl.ds(base, SEG1)],
                           ei_v.at[:, pl.ds(0, SEG1)], sem_a)
    ca2 = pltpu.async_copy(pq_hbm.at[0], p_v, sem_a)
    ca3 = pltpu.async_copy(pq_hbm.at[1], q_v, sem_a)
    cb = pltpu.async_copy(ei_hbm.at[:, pl.ds(base + SEG1, SEG2)],
                          ei_v.at[:, pl.ds(SEG1, SEG2)], sem_b)
    ca1.wait()
    ca2.wait()
    ca3.wait()

    init = (jnp.full((L,), jnp.inf, jnp.float32),
            jnp.full((L,), -jnp.inf, jnp.float32))

    @plsc.parallel_loop(0, SEG1 // L, unroll=4, carry=init)
    def _(i, carry):
        mn, mx = carry
        off = pl.multiple_of(i * L, L)
        si = ei_v[0, pl.ds(off, L)]
        di = ei_v[1, pl.ds(off, L)]
        sv = plsc.load_gather(p_v, [si]) + plsc.load_gather(q_v, [di])
        s_v[pl.ds(off, L)] = sv
        return jnp.minimum(mn, sv), jnp.maximum(mx, sv)

    mn1, mx1 = _
    cw1 = pltpu.async_copy(s_v.at[pl.ds(0, SEG1)],
                           s_hbm.at[0, pl.ds(base, SEG1)], sem_w)
    cb.wait()

    @plsc.parallel_loop(0, SEG2 // L, unroll=4, carry=(mn1, mx1))
    def _(i, carry):
        mn, mx = carry
        off = pl.multiple_of(SEG1 + i * L, L)
        si = ei_v[0, pl.ds(off, L)]
        di = ei_v[1, pl.ds(off, L)]
        sv = plsc.load_gather(p_v, [si]) + plsc.load_gather(q_v, [di])
        s_v[pl.ds(off, L)] = sv
        return jnp.minimum(mn, sv), jnp.maximum(mx, sv)

    mn, mx = _
    lane = lax.iota(jnp.int32, L)
    mm_v[...] = jnp.where(lane == 1, jnp.max(mx), jnp.min(mn))
    cw2 = pltpu.async_copy(s_v.at[pl.ds(SEG1, SEG2)],
                           s_hbm.at[0, pl.ds(base + SEG1, SEG2)], sem_w)
    pltpu.sync_copy(mm_v, mm_hbm.at[wid])
    cw1.wait()
    cw2.wait()


def kernel(h, edge_index, W, b):
    w2 = W.reshape(2, D)

    pq = pl.pallas_call(
        _pq_body,
        out_shape=jax.ShapeDtypeStruct((2, N), jnp.float32),
        in_specs=[
            pl.BlockSpec(),
            pl.BlockSpec(),
            pl.BlockSpec(memory_space=pltpu.SMEM),
        ],
    )(h, w2, b)

    sc = pl.kernel(
        _sc_body,
        out_type=(jax.ShapeDtypeStruct((1, E), jnp.float32),
                  jax.ShapeDtypeStruct((NW, L), jnp.float32)),
        mesh=plsc.VectorSubcoreMesh(core_axis_name="c", subcore_axis_name="s",
                                    num_cores=NC, num_subcores=NS),
        compiler_params=pltpu.CompilerParams(needs_layout_passes=False),
        scratch_types=[
            pltpu.VMEM((N,), jnp.float32),
            pltpu.VMEM((N,), jnp.float32),
            pltpu.VMEM((2, EW), jnp.int32),
            pltpu.VMEM((EW,), jnp.float32),
            pltpu.VMEM((L,), jnp.float32),
            pltpu.SemaphoreType.DMA,
            pltpu.SemaphoreType.DMA,
            pltpu.SemaphoreType.DMA,
        ],
    )
    s, mm = sc(pq, edge_index)

    o = pl.pallas_call(
        _norm_body,
        out_shape=jax.ShapeDtypeStruct((1, E), jnp.float32),
    )(s, mm)
    return o.reshape(E, 1)
